# baseline (speedup 1.0000x reference)
import jax
import jax.numpy as jnp
from jax import lax
from jax.experimental import pallas as pl
from jax.experimental.pallas import tpu as pltpu
from jax.experimental.pallas import tpu_sc as plsc

VOCAB_N = 100000
D = 1024
BATCH_N = 4
SEQ_N = 2048
TOK_TOTAL = BATCH_N * SEQ_N
EPS_LN = 1e-5

NC = 2
NS = 16
L = 16
NW = NC * NS
SPW = SEQ_N // NW
CH = 16
NSC = SPW // CH
NBLK = BATCH_N * NSC
NRING = 4
NJ = D // L


def _rsqrt_nr(x):
    i = plsc.bitcast(x, jnp.int32)
    i = jnp.int32(0x5F3759DF) - lax.shift_right_logical(i, 1)
    y = plsc.bitcast(i, jnp.float32)
    half = jnp.float32(0.5) * x
    for _ in range(3):
        y = y * (jnp.float32(1.5) - half * y * y)
    return y


def _emb_ln_body(x_hbm, word_hbm, pos_hbm, tok_hbm, gamma_hbm, beta_hbm,
                 out_hbm, idx_v, wbuf0, wbuf1, wbuf2, wbuf3, pbuf,
                 tok_v, gam_v, bet_v, stat_a, stat_m,
                 sem0, sem1, sem2, sem3, osem0, osem1, osem2, osem3, psem):
    wid = lax.axis_index("s") * NC + lax.axis_index("c")
    s0 = wid * SPW

    for bb in range(BATCH_N):
        pltpu.sync_copy(x_hbm.at[bb, pl.ds(s0, SPW)], idx_v.at[bb])
    pltpu.sync_copy(tok_hbm.at[0], tok_v)
    pltpu.sync_copy(gamma_hbm, gam_v)
    pltpu.sync_copy(beta_hbm, bet_v)

    wbufs = (wbuf0, wbuf1, wbuf2, wbuf3)
    sems = (sem0, sem1, sem2, sem3)
    osems = (osem0, osem1, osem2, osem3)
    inv_d = jnp.float32(1.0 / D)

    def gather_copy(k, d, b):
        sc = k // BATCH_N
        return pltpu.make_async_copy(
            word_hbm.at[idx_v.at[b, pl.ds(pl.multiple_of(sc * CH, CH), CH)]],
            wbufs[d], sems[d])

    def pos_copy(sc):
        return pltpu.make_async_copy(
            pos_hbm.at[pl.ds(s0 + sc * CH, CH)], pbuf, psem)

    def out_copy(k, d, b):
        sc = k // BATCH_N
        base = pl.multiple_of(b * SEQ_N + s0 + sc * CH, CH)
        return pltpu.make_async_copy(
            wbufs[d], out_hbm.at[pl.ds(base, CH)], osems[d])

    pos_copy(0).start()
    gather_copy(0, 0, 0).start()

    @pl.loop(0, NBLK, step=NRING)
    def kloop(k0):
        for d in range(NRING):
            k = k0 + d
            sc = k // BATCH_N
            b = d

            dn = (d + 1) % NRING

            @pl.when(k < NBLK - 1)
            def _prefetch(k=k, dn=dn):
                @pl.when(k >= NRING - 1)
                def _drain(k=k, dn=dn):
                    out_copy(k - (NRING - 1), dn, dn).wait()
                gather_copy(k + 1, dn, (d + 1) % BATCH_N).start()

            if b == 0:
                pos_copy(sc).wait()

                @plsc.parallel_loop(0, CH)
                def _fold_tok(r):
                    for j in range(NJ):
                        sl = pl.ds(j * L, L)
                        pbuf[r, sl] = pbuf[r, sl] + tok_v[sl]

            gather_copy(k, d, b).wait()
            wb = wbufs[d]

            @plsc.parallel_loop(0, CH)
            def _row(r, wb=wb):
                acc = [jnp.zeros((L,), jnp.float32) for _ in range(4)]
                acc2 = [jnp.zeros((L,), jnp.float32) for _ in range(4)]
                for j in range(NJ):
                    sl = pl.ds(j * L, L)
                    h = wb[r, sl] + pbuf[r, sl]
                    wb[r, sl] = h
                    m = j % 4
                    acc[m] = acc[m] + h
                    acc2[m] = acc2[m] + h * h
                s1 = jnp.sum((acc[0] + acc[1]) + (acc[2] + acc[3]))
                s2 = jnp.sum((acc2[0] + acc2[1]) + (acc2[2] + acc2[3]))
                mean = s1 * inv_d
                var = s2 * inv_d - mean * mean
                rstd = _rsqrt_nr(jnp.full((L,), var + EPS_LN, jnp.float32))
                stat_a[r] = rstd
                stat_m[r] = jnp.full((L,), mean, jnp.float32) * rstd

            if b == BATCH_N - 1:
                @pl.when(sc < NSC - 1)
                def _prefetch_pos(sc=sc):
                    pos_copy(sc + 1).start()

            a_r = [stat_a[i] for i in range(CH)]
            m_r = [stat_m[i] for i in range(CH)]

            @plsc.parallel_loop(0, NJ)
            def _col(j, wb=wb, a_r=a_r, m_r=m_r):
                sl = pl.ds(j * L, L)
                g = gam_v[sl]
                bb2 = bet_v[sl]
                for i in range(CH):
                    h = wb[i, sl]
                    wb[i, sl] = (h * a_r[i] - m_r[i]) * g + bb2

            out_copy(k, d, b).start()

    for d in range(NRING):
        out_copy(NBLK - NRING + d, d, d).wait()


@jax.jit
def _emb_ln(x, word_emb, pos_emb, tok_emb, gamma, beta):
    mesh = plsc.VectorSubcoreMesh(
        core_axis_name="c", subcore_axis_name="s",
        num_cores=NC, num_subcores=NS)
    return pl.kernel(
        _emb_ln_body,
        out_type=jax.ShapeDtypeStruct((TOK_TOTAL, D), jnp.float32),
        mesh=mesh,
        compiler_params=pltpu.CompilerParams(needs_layout_passes=False),
        scratch_types=[
            pltpu.VMEM((BATCH_N, SPW), jnp.int32),
            pltpu.VMEM((CH, D), jnp.float32),
            pltpu.VMEM((CH, D), jnp.float32),
            pltpu.VMEM((CH, D), jnp.float32),
            pltpu.VMEM((CH, D), jnp.float32),
            pltpu.VMEM((CH, D), jnp.float32),
            pltpu.VMEM((D,), jnp.float32),
            pltpu.VMEM((D,), jnp.float32),
            pltpu.VMEM((D,), jnp.float32),
            pltpu.VMEM((CH, L), jnp.float32),
            pltpu.VMEM((CH, L), jnp.float32),
            pltpu.SemaphoreType.DMA,
            pltpu.SemaphoreType.DMA,
            pltpu.SemaphoreType.DMA,
            pltpu.SemaphoreType.DMA,
            pltpu.SemaphoreType.DMA,
            pltpu.SemaphoreType.DMA,
            pltpu.SemaphoreType.DMA,
            pltpu.SemaphoreType.DMA,
            pltpu.SemaphoreType.DMA,
        ],
    )(x, word_emb, pos_emb, tok_emb, gamma, beta)


def kernel(x, word_emb, pos_emb, tok_emb, gamma, beta):
    xi = x.astype(jnp.int32)
    out = _emb_ln(xi, word_emb, pos_emb, tok_emb, gamma, beta)
    return out.reshape(BATCH_N, SEQ_N, D)


# speedup vs baseline: 1.0921x; 1.0921x over previous
import jax
import jax.numpy as jnp
from jax import lax
from jax.experimental import pallas as pl
from jax.experimental.pallas import tpu as pltpu
from jax.experimental.pallas import tpu_sc as plsc

VOCAB_N = 100000
D = 1024
BATCH_N = 4
SEQ_N = 2048
TOK_TOTAL = BATCH_N * SEQ_N
EPS_LN = 1e-5

NC = 2
NS = 16
L = 16
NW = NC * NS
SPW = SEQ_N // NW
CH = 32
NSC = SPW // CH
NBLK = BATCH_N * NSC
NJ = D // L
RGRP = 16


def _rsqrt_nr(x):
    i = plsc.bitcast(x, jnp.int32)
    i = jnp.int32(0x5F3759DF) - lax.shift_right_logical(i, 1)
    y = plsc.bitcast(i, jnp.float32)
    half = jnp.float32(0.5) * x
    for _ in range(3):
        y = y * (jnp.float32(1.5) - half * y * y)
    return y


def _emb_ln_body(x_hbm, word_hbm, pos_hbm, tok_hbm, gamma_hbm, beta_hbm,
                 out_hbm, idx_v, wbuf0, wbuf1, pbuf, tok_v, gam_v, bet_v,
                 stat_a, stat_m, sem0, sem1, osem0, osem1, psem):
    wid = lax.axis_index("s") * NC + lax.axis_index("c")
    s0 = wid * SPW

    for bb in range(BATCH_N):
        pltpu.sync_copy(x_hbm.at[bb, pl.ds(s0, SPW)], idx_v.at[bb])
    pltpu.sync_copy(tok_hbm.at[0], tok_v)
    pltpu.sync_copy(gamma_hbm, gam_v)
    pltpu.sync_copy(beta_hbm, bet_v)

    wbufs = (wbuf0, wbuf1)
    sems = (sem0, sem1)
    osems = (osem0, osem1)
    inv_d = jnp.float32(1.0 / D)

    def gather_copy(k, d):
        sc = k // BATCH_N
        b = lax.rem(k, BATCH_N)
        return pltpu.make_async_copy(
            word_hbm.at[idx_v.at[b, pl.ds(pl.multiple_of(sc * CH, CH), CH)]],
            wbufs[d], sems[d])

    def pos_copy(sc):
        return pltpu.make_async_copy(
            pos_hbm.at[pl.ds(s0 + sc * CH, CH)], pbuf, psem)

    def out_copy(k, d):
        sc = k // BATCH_N
        b = lax.rem(k, BATCH_N)
        base = pl.multiple_of(b * SEQ_N + s0 + sc * CH, CH)
        return pltpu.make_async_copy(
            wbufs[d], out_hbm.at[pl.ds(base, CH)], osems[d])

    pos_copy(0).start()
    gather_copy(0, 0).start()

    @pl.loop(0, NBLK, step=2)
    def kloop(k0):
        for d in range(2):
            k = k0 + d
            sc = k // BATCH_N
            b = lax.rem(k, BATCH_N)
            first_b = b == 0

            @pl.when(first_b)
            def _wait_pos(sc=sc):
                pos_copy(sc).wait()

                @plsc.parallel_loop(0, CH)
                def _fold_tok(r):
                    for j in range(NJ):
                        sl = pl.ds(j * L, L)
                        pbuf[r, sl] = pbuf[r, sl] + tok_v[sl]

            gather_copy(k, d).wait()
            wb = wbufs[d]

            @plsc.parallel_loop(0, CH)
            def _row(r, wb=wb):
                acc0 = jnp.zeros((L,), jnp.float32)
                acc1 = jnp.zeros((L,), jnp.float32)
                acc2_0 = jnp.zeros((L,), jnp.float32)
                acc2_1 = jnp.zeros((L,), jnp.float32)
                for j in range(NJ):
                    sl = pl.ds(j * L, L)
                    h = wb[r, sl] + pbuf[r, sl]
                    wb[r, sl] = h
                    if j % 2 == 0:
                        acc0 = acc0 + h
                        acc2_0 = acc2_0 + h * h
                    else:
                        acc1 = acc1 + h
                        acc2_1 = acc2_1 + h * h
                s1 = jnp.sum(acc0 + acc1)
                s2 = jnp.sum(acc2_0 + acc2_1)
                mean = s1 * inv_d
                var = s2 * inv_d - mean * mean
                rstd = _rsqrt_nr(jnp.full((L,), var + EPS_LN, jnp.float32))
                stat_a[r] = rstd
                stat_m[r] = jnp.full((L,), mean, jnp.float32) * rstd

            @pl.when(k < NBLK - 1)
            def _prefetch(k=k, d=d):
                @pl.when(k >= 1)
                def _drain(k=k, d=d):
                    out_copy(k - 1, 1 - d).wait()
                gather_copy(k + 1, 1 - d).start()

            @pl.when((b == BATCH_N - 1) & (sc < NSC - 1))
            def _prefetch_pos(sc=sc):
                pos_copy(sc + 1).start()

            for r0 in range(0, CH, RGRP):
                a_r = [stat_a[r0 + i] for i in range(RGRP)]
                m_r = [stat_m[r0 + i] for i in range(RGRP)]

                @plsc.parallel_loop(0, NJ)
                def _col(j, wb=wb, a_r=a_r, m_r=m_r, r0=r0):
                    sl = pl.ds(j * L, L)
                    g = gam_v[sl]
                    bb2 = bet_v[sl]
                    for i in range(RGRP):
                        h = wb[r0 + i, sl]
                        wb[r0 + i, sl] = (h * a_r[i] - m_r[i]) * g + bb2

            out_copy(k, d).start()

    for d in range(2):
        out_copy(NBLK - 2 + d, d).wait()


@jax.jit
def _emb_ln(x, word_emb, pos_emb, tok_emb, gamma, beta):
    mesh = plsc.VectorSubcoreMesh(
        core_axis_name="c", subcore_axis_name="s",
        num_cores=NC, num_subcores=NS)
    return pl.kernel(
        _emb_ln_body,
        out_type=jax.ShapeDtypeStruct((TOK_TOTAL, D), jnp.float32),
        mesh=mesh,
        compiler_params=pltpu.CompilerParams(needs_layout_passes=False),
        scratch_types=[
            pltpu.VMEM((BATCH_N, SPW), jnp.int32),
            pltpu.VMEM((CH, D), jnp.float32),
            pltpu.VMEM((CH, D), jnp.float32),
            pltpu.VMEM((CH, D), jnp.float32),
            pltpu.VMEM((D,), jnp.float32),
            pltpu.VMEM((D,), jnp.float32),
            pltpu.VMEM((D,), jnp.float32),
            pltpu.VMEM((CH, L), jnp.float32),
            pltpu.VMEM((CH, L), jnp.float32),
            pltpu.SemaphoreType.DMA,
            pltpu.SemaphoreType.DMA,
            pltpu.SemaphoreType.DMA,
            pltpu.SemaphoreType.DMA,
            pltpu.SemaphoreType.DMA,
        ],
    )(x, word_emb, pos_emb, tok_emb, gamma, beta)


def kernel(x, word_emb, pos_emb, tok_emb, gamma, beta):
    xi = x.astype(jnp.int32)
    out = _emb_ln(xi, word_emb, pos_emb, tok_emb, gamma, beta)
    return out.reshape(BATCH_N, SEQ_N, D)


# speedup vs baseline: 1.1235x; 1.0287x over previous
import jax
import jax.numpy as jnp
from jax import lax
from jax.experimental import pallas as pl
from jax.experimental.pallas import tpu as pltpu
from jax.experimental.pallas import tpu_sc as plsc

VOCAB_N = 100000
D = 1024
BATCH_N = 4
SEQ_N = 2048
TOK_TOTAL = BATCH_N * SEQ_N
EPS_LN = 1e-5

NC = 2
NS = 16
L = 16
NW = NC * NS
SPW = SEQ_N // NW
CH = 32
NSC = SPW // CH
NBLK = BATCH_N * NSC
NJ = D // L
RGRP = 16


def _rsqrt_nr(x):
    i = plsc.bitcast(x, jnp.int32)
    i = jnp.int32(0x5F3759DF) - lax.shift_right_logical(i, 1)
    y = plsc.bitcast(i, jnp.float32)
    half = jnp.float32(0.5) * x
    for _ in range(3):
        y = y * (jnp.float32(1.5) - half * y * y)
    return y


def _emb_ln_body(x_hbm, word_hbm, pos_hbm, tok_hbm, gamma_hbm, beta_hbm,
                 out_hbm, idx_v, wbuf0, wbuf1, pbuf, tok_v, gam_v, bet_v,
                 stat_a, stat_m, sem0, sem1, osem0, osem1, psem):
    wid = lax.axis_index("s") * NC + lax.axis_index("c")
    s0 = wid * SPW

    for bb in range(BATCH_N):
        pltpu.sync_copy(x_hbm.at[bb, pl.ds(s0, SPW)], idx_v.at[bb])
    pltpu.sync_copy(tok_hbm.at[0], tok_v)
    pltpu.sync_copy(gamma_hbm, gam_v)
    pltpu.sync_copy(beta_hbm, bet_v)

    wbufs = (wbuf0, wbuf1)
    sems = (sem0, sem1)
    osems = (osem0, osem1)
    inv_d = jnp.float32(1.0 / D)

    def gather_copy(k, d):
        sc = k // BATCH_N
        b = lax.rem(k, BATCH_N)
        return pltpu.make_async_copy(
            word_hbm.at[idx_v.at[b, pl.ds(pl.multiple_of(sc * CH, CH), CH)]],
            wbufs[d], sems[d])

    def pos_copy(sc):
        return pltpu.make_async_copy(
            pos_hbm.at[pl.ds(s0 + sc * CH, CH)], pbuf, psem)

    def out_copy(k, d):
        sc = k // BATCH_N
        b = lax.rem(k, BATCH_N)
        base = pl.multiple_of(b * SEQ_N + s0 + sc * CH, CH)
        return pltpu.make_async_copy(
            wbufs[d], out_hbm.at[pl.ds(base, CH)], osems[d])

    pos_copy(0).start()
    gather_copy(0, 0).start()

    @pl.loop(0, NBLK, step=2)
    def kloop(k0):
        for d in range(2):
            k = k0 + d
            sc = k // BATCH_N
            b = lax.rem(k, BATCH_N)
            first_b = b == 0

            @pl.when(first_b)
            def _wait_pos(sc=sc):
                pos_copy(sc).wait()

                @plsc.parallel_loop(0, CH)
                def _fold_tok(r):
                    for j in range(NJ):
                        sl = pl.ds(j * L, L)
                        pbuf[r, sl] = pbuf[r, sl] + tok_v[sl]

            gather_copy(k, d).wait()
            wb = wbufs[d]

            @plsc.parallel_loop(0, CH)
            def _row(r, wb=wb):
                acc = [jnp.zeros((L,), jnp.float32) for _ in range(4)]
                acc2 = [jnp.zeros((L,), jnp.float32) for _ in range(4)]
                for j in range(NJ):
                    sl = pl.ds(j * L, L)
                    h = wb[r, sl] + pbuf[r, sl]
                    wb[r, sl] = h
                    m = j % 4
                    acc[m] = acc[m] + h
                    acc2[m] = acc2[m] + h * h
                s1 = jnp.sum((acc[0] + acc[1]) + (acc[2] + acc[3]))
                s2 = jnp.sum((acc2[0] + acc2[1]) + (acc2[2] + acc2[3]))
                mean = s1 * inv_d
                var = s2 * inv_d - mean * mean
                rstd = _rsqrt_nr(jnp.full((L,), var + EPS_LN, jnp.float32))
                stat_a[r] = rstd
                stat_m[r] = jnp.full((L,), mean, jnp.float32) * rstd

            @pl.when(k < NBLK - 1)
            def _prefetch(k=k, d=d):
                @pl.when(k >= 1)
                def _drain(k=k, d=d):
                    out_copy(k - 1, 1 - d).wait()
                gather_copy(k + 1, 1 - d).start()

            @pl.when((b == BATCH_N - 1) & (sc < NSC - 1))
            def _prefetch_pos(sc=sc):
                pos_copy(sc + 1).start()

            for r0 in range(0, CH, RGRP):
                a_r = [stat_a[r0 + i] for i in range(RGRP)]
                m_r = [stat_m[r0 + i] for i in range(RGRP)]

                @plsc.parallel_loop(0, NJ)
                def _col(j, wb=wb, a_r=a_r, m_r=m_r, r0=r0):
                    sl = pl.ds(j * L, L)
                    g = gam_v[sl]
                    bb2 = bet_v[sl]
                    for i in range(RGRP):
                        h = wb[r0 + i, sl]
                        wb[r0 + i, sl] = (h * a_r[i] - m_r[i]) * g + bb2

            out_copy(k, d).start()

    for d in range(2):
        out_copy(NBLK - 2 + d, d).wait()


@jax.jit
def _emb_ln(x, word_emb, pos_emb, tok_emb, gamma, beta):
    mesh = plsc.VectorSubcoreMesh(
        core_axis_name="c", subcore_axis_name="s",
        num_cores=NC, num_subcores=NS)
    return pl.kernel(
        _emb_ln_body,
        out_type=jax.ShapeDtypeStruct((TOK_TOTAL, D), jnp.float32),
        mesh=mesh,
        compiler_params=pltpu.CompilerParams(needs_layout_passes=False),
        scratch_types=[
            pltpu.VMEM((BATCH_N, SPW), jnp.int32),
            pltpu.VMEM((CH, D), jnp.float32),
            pltpu.VMEM((CH, D), jnp.float32),
            pltpu.VMEM((CH, D), jnp.float32),
            pltpu.VMEM((D,), jnp.float32),
            pltpu.VMEM((D,), jnp.float32),
            pltpu.VMEM((D,), jnp.float32),
            pltpu.VMEM((CH, L), jnp.float32),
            pltpu.VMEM((CH, L), jnp.float32),
            pltpu.SemaphoreType.DMA,
            pltpu.SemaphoreType.DMA,
            pltpu.SemaphoreType.DMA,
            pltpu.SemaphoreType.DMA,
            pltpu.SemaphoreType.DMA,
        ],
    )(x, word_emb, pos_emb, tok_emb, gamma, beta)


def kernel(x, word_emb, pos_emb, tok_emb, gamma, beta):
    xi = x.astype(jnp.int32)
    out = _emb_ln(xi, word_emb, pos_emb, tok_emb, gamma, beta)
    return out.reshape(BATCH_N, SEQ_N, D)
